# Initial kernel scaffold; baseline (speedup 1.0000x reference)
import jax
import jax.numpy as jnp
from jax.experimental import pallas as pl


def kernel(embedding_weight, pos_ix, neg_ix):
    raise NotImplementedError("write your pallas kernel here")


# speedup vs baseline: 1.2669x; 1.2669x over previous
import functools

import jax
import jax.numpy as jnp
from jax import lax
from jax.experimental import pallas as pl
from jax.experimental.pallas import tpu as pltpu
from jax.experimental.pallas import tpu_sc as plsc

_V = 10000
_D = 128
_NC, _NS, _L = 2, 16, 16
_NW = _NC * _NS
_NPAIRS = 320000
_T = 2 * _NPAIRS
_PW = _T // _NW
_C = 80
_J = _PW // _C
_POS_M = 0.1
_NEG_M = 1.0


def _rsqrt16(t):
    i = lax.bitcast_convert_type(t, jnp.int32)
    i = jnp.int32(0x5F3759DF) - lax.shift_right_logical(i, 1)
    y = lax.bitcast_convert_type(i, jnp.float32)
    for _ in range(3):
        y = y * (1.5 - 0.5 * t * y * y)
    return y


def _make_sc_fn():
    mesh = plsc.VectorSubcoreMesh(core_axis_name="c", subcore_axis_name="s")

    @functools.partial(
        pl.kernel,
        out_type=jax.ShapeDtypeStruct((_NW, _L), jnp.float32),
        mesh=mesh,
        compiler_params=pltpu.CompilerParams(needs_layout_passes=False),
        scratch_types=[
            pltpu.VMEM((_J, _C), jnp.int32),
            pltpu.VMEM((_J, _C), jnp.int32),
            pltpu.VMEM((2, _C, _D), jnp.float32),
            pltpu.VMEM((2, _C, _D), jnp.float32),
            pltpu.VMEM((_L,), jnp.float32),
            pltpu.SemaphoreType.DMA,
            pltpu.SemaphoreType.DMA,
            pltpu.SemaphoreType.DMA,
            pltpu.SemaphoreType.DMA,
        ],
    )
    def sc_fn(ix0_hbm, ix1_hbm, table_hbm, out_hbm,
              ix0_v, ix1_v, rows0_v, rows1_v, res_v,
              sa0, sa1, sb0, sb1):
        wid = lax.axis_index("s") * _NC + lax.axis_index("c")
        iota = lax.iota(jnp.int32, _L)
        is_pos = wid < (_NW // 2)
        sgn = jnp.where(is_pos, 1.0, -1.0).astype(jnp.float32)
        off_m = jnp.where(is_pos, _POS_M, _NEG_M).astype(jnp.float32) * sgn

        pltpu.sync_copy(ix0_hbm.at[wid], ix0_v)
        pltpu.sync_copy(ix1_hbm.at[wid], ix1_v)

        sems = ((sa0, sa1), (sb0, sb1))

        def start_gather(j, slot):
            s0, s1 = sems[slot]
            c0 = pltpu.async_copy(table_hbm.at[ix0_v.at[j]],
                                  rows0_v.at[slot], s0)
            c1 = pltpu.async_copy(table_hbm.at[ix1_v.at[j]],
                                  rows1_v.at[slot], s1)
            return c0, c1

        def wait_gather(j, slot):
            s0, s1 = sems[slot]
            pltpu.make_async_copy(table_hbm.at[ix0_v.at[j]],
                                  rows0_v.at[slot], s0).wait()
            pltpu.make_async_copy(table_hbm.at[ix1_v.at[j]],
                                  rows1_v.at[slot], s1).wait()

        def compute(j, slot, acc):
            r0 = rows0_v.at[slot]
            r1 = rows1_v.at[slot]

            def group_body(g, acc_g):
                pid = g * _L + iota

                def feat_body(k, s):
                    for u in range(8):
                        fv = jnp.full((_L,), k * 8 + u, dtype=jnp.int32)
                        a = plsc.load_gather(r0, [pid, fv])
                        b = plsc.load_gather(r1, [pid, fv])
                        dd = a - b
                        s = s + dd * dd
                    return s

                s = lax.fori_loop(0, _D // 8, feat_body,
                                  jnp.zeros((_L,), jnp.float32))
                t = s + 1e-12
                d = t * _rsqrt16(t)
                h = jnp.maximum(sgn * d - off_m, 0.0)
                return acc_g + h * h

            return lax.fori_loop(0, _C // _L, group_body, acc)

        start_gather(0, 0)

        def pair_body(p, acc):
            j0 = 2 * p
            start_gather(j0 + 1, 1)
            wait_gather(j0, 0)
            acc = compute(j0, 0, acc)

            @pl.when(p + 1 < _J // 2)
            def _():
                start_gather(j0 + 2, 0)

            wait_gather(j0 + 1, 1)
            acc = compute(j0 + 1, 1, acc)
            return acc

        acc = lax.fori_loop(0, _J // 2, pair_body,
                            jnp.zeros((_L,), jnp.float32))
        res_v[...] = acc
        pltpu.sync_copy(res_v, out_hbm.at[wid])

    return sc_fn


def kernel(embedding_weight, pos_ix, neg_ix):
    ix = jnp.concatenate([pos_ix, neg_ix], axis=1)
    ix0 = ix[0].reshape(_NW, _J, _C)
    ix1 = ix[1].reshape(_NW, _J, _C)
    partials = _make_sc_fn()(ix0, ix1, embedding_weight)
    return jnp.sum(partials)
